# Initial kernel scaffold; baseline (speedup 1.0000x reference)
import jax
import jax.numpy as jnp
from jax.experimental import pallas as pl


def kernel(preds, ground_truth):
    raise NotImplementedError("write your pallas kernel here")


# speedup vs baseline: 4.8607x; 4.8607x over previous
import functools

import jax
import jax.numpy as jnp
from jax import lax
from jax.experimental import pallas as pl
from jax.experimental.pallas import tpu as pltpu
from jax.experimental.pallas import tpu_sc as plsc

EPSILON = 1e-6
BATCH = 1024
NUM_VARS = 64
VAR_SIZE = 256
NUM_ANCHORS = 16
LANES = 16
NUM_WORKERS = 32
ROWS_PER_WORKER = BATCH // NUM_WORKERS

_mesh = plsc.VectorSubcoreMesh(core_axis_name="c", subcore_axis_name="s")


@functools.partial(
    pl.kernel,
    out_type=jax.ShapeDtypeStruct((BATCH, NUM_VARS, VAR_SIZE), jnp.float32),
    mesh=_mesh,
    scratch_types=[
        pltpu.VMEM((NUM_VARS, VAR_SIZE), jnp.float32),
        pltpu.VMEM((NUM_ANCHORS, VAR_SIZE), jnp.float32),
        pltpu.SemaphoreType.DMA,
    ],
)
def _sc_grad_optim(preds_hbm, gt_hbm, out_hbm, pbuf, gbuf, sem):
    c = lax.axis_index("c")
    s = lax.axis_index("s")
    wid = s * 2 + c
    base = wid * ROWS_PER_WORKER

    def one_row(i, carry):
        row = base + i
        cp_p = pltpu.make_async_copy(preds_hbm.at[row], pbuf, sem)
        cp_g = pltpu.make_async_copy(
            gt_hbm.at[row, pl.ds(2 * NUM_ANCHORS, NUM_ANCHORS)], gbuf, sem
        )
        cp_p.start()
        cp_g.start()
        cp_p.wait()
        cp_g.wait()
        for a in range(NUM_ANCHORS):
            def chunk(j, carry2):
                off = pl.multiple_of(j * LANES, LANES)
                g = gbuf[a, pl.ds(off, LANES)]
                av = pbuf[a, pl.ds(off, LANES)]
                m1 = (pbuf[a + 16, pl.ds(off, LANES)] + EPSILON) - g
                m2 = (pbuf[a + 48, pl.ds(off, LANES)] - EPSILON) - g
                pbuf[a, pl.ds(off, LANES)] = jnp.maximum(jnp.maximum(av, m1), m2)
                return carry2

            lax.fori_loop(0, VAR_SIZE // LANES, chunk, 0)
        pltpu.sync_copy(pbuf, out_hbm.at[row])
        return carry

    lax.fori_loop(0, ROWS_PER_WORKER, one_row, 0)


def kernel(preds, ground_truth):
    return _sc_grad_optim(preds, ground_truth)
